# Initial kernel scaffold; baseline (speedup 1.0000x reference)
import jax
import jax.numpy as jnp
from jax.experimental import pallas as pl


def kernel(x, edge_index, W1, b1, W2, b2):
    raise NotImplementedError("write your pallas kernel here")


# speedup vs baseline: 7.7202x; 7.7202x over previous
import functools

import jax
import jax.numpy as jnp
from jax import lax
from jax.experimental import pallas as pl
from jax.experimental.pallas import tpu as pltpu
from jax.experimental.pallas import tpu_sc as plsc

_N = 10000
_E = 160000
_D_IN = 256
_D_HID = 512
_D_OUT = 256
_NC = 2
_NS = 16
_CHUNK = 128
_NPAD = 10112
_RPT = _NPAD // _NS
_TRASH = _NPAD - _N
_EPAD = 163840
_DEG_NCHUNK = _EPAD // (_NC * _NS * _CHUNK)
_AGG_NCHUNK = _EPAD // (_NS * _CHUNK)
_HALF = 128
_W = 16

_SC_PARAMS = pltpu.CompilerParams(use_tc_tiling_on_sc=False)


def _mesh():
    return plsc.VectorSubcoreMesh(
        core_axis_name="c", subcore_axis_name="s", num_cores=_NC, num_subcores=_NS
    )


def _sc_hist(sidx, ones_tile, zeros1):
    @functools.partial(
        pl.kernel,
        mesh=_mesh(),
        out_type=[jax.ShapeDtypeStruct((_NPAD, _W), jnp.float32) for _ in range(2)],
        scratch_types=[
            pltpu.VMEM((_DEG_NCHUNK, _CHUNK), jnp.int32),
            pltpu.VMEM((_CHUNK, _W), jnp.float32),
            pltpu.VMEM_SHARED((_NPAD, _W), jnp.float32),
        ],
        compiler_params=_SC_PARAMS,
    )
    def k(sidx_h, ones_h, z_h, o0_h, o1_h, sv, buf, acc):
        c = lax.axis_index("c")
        s = lax.axis_index("s")
        tid = c * _NS + s
        slc = pl.ds(s * _RPT, _RPT)
        pltpu.sync_copy(z_h, acc.at[slc])
        pltpu.sync_copy(ones_h, buf)
        pltpu.sync_copy(sidx_h.at[tid], sv)
        plsc.subcore_barrier()

        @pl.loop(0, _DEG_NCHUNK)
        def _(j):
            pltpu.sync_copy(buf, acc.at[sv.at[j]], add=True)

        plsc.subcore_barrier()

        @pl.when(c == 0)
        def _():
            pltpu.sync_copy(acc.at[slc], o0_h.at[slc])

        @pl.when(c == 1)
        def _():
            pltpu.sync_copy(acc.at[slc], o1_h.at[slc])

    return k(sidx, ones_tile, zeros1)


def _sc_gather_scatter_w1(table, gidx, sidx, zeros1):
    @functools.partial(
        pl.kernel,
        mesh=_mesh(),
        out_type=[jax.ShapeDtypeStruct((_NPAD, _W), jnp.float32) for _ in range(2)],
        scratch_types=[
            pltpu.VMEM((_DEG_NCHUNK, _CHUNK), jnp.int32),
            pltpu.VMEM((_DEG_NCHUNK, _CHUNK), jnp.int32),
            pltpu.VMEM((_CHUNK, _W), jnp.float32),
            pltpu.VMEM_SHARED((_NPAD, _W), jnp.float32),
            pltpu.SemaphoreType.DMA,
        ],
        compiler_params=_SC_PARAMS,
    )
    def k(table_h, gidx_h, sidx_h, z_h, o0_h, o1_h, gv, sv, buf, acc, sem):
        c = lax.axis_index("c")
        s = lax.axis_index("s")
        tid = c * _NS + s
        slc = pl.ds(s * _RPT, _RPT)
        pltpu.sync_copy(z_h, acc.at[slc])
        pltpu.sync_copy(gidx_h.at[tid], gv)
        pltpu.sync_copy(sidx_h.at[tid], sv)
        plsc.subcore_barrier()

        @pl.loop(0, _DEG_NCHUNK)
        def _(j):
            pltpu.async_copy(table_h.at[gv.at[j]], buf, sem).wait()
            pltpu.sync_copy(buf, acc.at[sv.at[j]], add=True)

        plsc.subcore_barrier()

        @pl.when(c == 0)
        def _():
            pltpu.sync_copy(acc.at[slc], o0_h.at[slc])

        @pl.when(c == 1)
        def _():
            pltpu.sync_copy(acc.at[slc], o1_h.at[slc])

    return k(table, gidx, sidx, zeros1)


def _sc_agg(t0, t1, gidx, sidx, zerosd):
    @functools.partial(
        pl.kernel,
        mesh=_mesh(),
        out_type=[jax.ShapeDtypeStruct((_NPAD, _HALF), jnp.float32) for _ in range(2)],
        scratch_types=[
            pltpu.VMEM((_AGG_NCHUNK, _CHUNK), jnp.int32),
            pltpu.VMEM((_AGG_NCHUNK, _CHUNK), jnp.int32),
            pltpu.VMEM((_CHUNK, _HALF), jnp.float32),
            pltpu.VMEM_SHARED((_NPAD, _HALF), jnp.float32),
            pltpu.SemaphoreType.DMA,
        ],
    )
    def k(t0_h, t1_h, gidx_h, sidx_h, z_h, o0_h, o1_h, gv, sv, buf, acc, sem):
        c = lax.axis_index("c")
        s = lax.axis_index("s")
        slc = pl.ds(s * _RPT, _RPT)
        pltpu.sync_copy(z_h, acc.at[slc])
        pltpu.sync_copy(gidx_h.at[s], gv)
        pltpu.sync_copy(sidx_h.at[s], sv)
        plsc.subcore_barrier()

        def edge_loop(th):
            @pl.loop(0, _AGG_NCHUNK)
            def _(j):
                pltpu.async_copy(th.at[gv.at[j]], buf, sem).wait()
                pltpu.sync_copy(buf, acc.at[sv.at[j]], add=True)

        @pl.when(c == 0)
        def _():
            edge_loop(t0_h)

        @pl.when(c == 1)
        def _():
            edge_loop(t1_h)

        plsc.subcore_barrier()

        @pl.when(c == 0)
        def _():
            pltpu.sync_copy(acc.at[slc], o0_h.at[slc])

        @pl.when(c == 1)
        def _():
            pltpu.sync_copy(acc.at[slc], o1_h.at[slc])

    return k(t0, t1, gidx, sidx, zerosd)


def _tc_rsqrt_sum(a, b):
    def body(a_ref, b_ref, o_ref):
        sm = a_ref[...] + b_ref[...]
        o_ref[...] = jnp.where(sm > 0, lax.rsqrt(sm), 0.0)

    return pl.pallas_call(
        body, out_shape=jax.ShapeDtypeStruct(a.shape, jnp.float32)
    )(a, b)


def _tc_colscale(c0, c1, xpad):
    r = 128
    grid = _NPAD // r

    def body(c0_ref, c1_ref, x_ref, xs_ref, dc_ref):
        sm = c0_ref[...] + c1_ref[...]
        dc = jnp.where(sm > 0, lax.rsqrt(sm), 0.0)[:, :1]
        dc_ref[...] = dc
        xs_ref[...] = x_ref[...] * dc

    return pl.pallas_call(
        body,
        grid=(grid,),
        in_specs=[
            pl.BlockSpec((r, _W), lambda i: (i, 0)),
            pl.BlockSpec((r, _W), lambda i: (i, 0)),
            pl.BlockSpec((r, _D_IN), lambda i: (i, 0)),
        ],
        out_specs=[
            pl.BlockSpec((r, _D_IN), lambda i: (i, 0)),
            pl.BlockSpec((r, 1), lambda i: (i, 0)),
        ],
        out_shape=[
            jax.ShapeDtypeStruct((_NPAD, _D_IN), jnp.float32),
            jax.ShapeDtypeStruct((_NPAD, 1), jnp.float32),
        ],
    )(c0, c1, xpad)


def _tc_mlp(a1, dr, w1, b1, w2, dc):
    r = 128
    grid = _NPAD // r

    def body(a_ref, dr_ref, w1_ref, b1_ref, w2_ref, dc_ref, o_ref):
        a = a_ref[...] * dr_ref[...]
        h = jnp.dot(a, w1_ref[...], preferred_element_type=jnp.float32)
        h = jnp.maximum(h + b1_ref[...], 0.0)
        m2 = jnp.dot(h, w2_ref[...], preferred_element_type=jnp.float32)
        o_ref[...] = m2 * dc_ref[...]

    return pl.pallas_call(
        body,
        grid=(grid,),
        in_specs=[
            pl.BlockSpec((r, _D_IN), lambda i: (i, 0)),
            pl.BlockSpec((r, 1), lambda i: (i, 0)),
            pl.BlockSpec((_D_IN, _D_HID), lambda i: (0, 0)),
            pl.BlockSpec((1, _D_HID), lambda i: (0, 0)),
            pl.BlockSpec((_D_HID, _D_OUT), lambda i: (0, 0)),
            pl.BlockSpec((r, 1), lambda i: (i, 0)),
        ],
        out_specs=pl.BlockSpec((r, _D_OUT), lambda i: (i, 0)),
        out_shape=jax.ShapeDtypeStruct((_NPAD, _D_OUT), jnp.float32),
    )(a1, dr, w1, b1, w2, dc)


def _tc_logsoftmax(a2, dr, b2):
    r = 128
    grid = _NPAD // r

    def body(a_ref, dr_ref, b2_ref, o_ref):
        v = a_ref[...] * dr_ref[...] + b2_ref[...]
        m = jnp.max(v, axis=-1, keepdims=True)
        z = v - m
        o_ref[...] = z - jnp.log(jnp.sum(jnp.exp(z), axis=-1, keepdims=True))

    return pl.pallas_call(
        body,
        grid=(grid,),
        in_specs=[
            pl.BlockSpec((r, _D_OUT), lambda i: (i, 0)),
            pl.BlockSpec((r, 1), lambda i: (i, 0)),
            pl.BlockSpec((1, _D_OUT), lambda i: (0, 0)),
        ],
        out_specs=pl.BlockSpec((r, _D_OUT), lambda i: (i, 0)),
        out_shape=jax.ShapeDtypeStruct((_NPAD, _D_OUT), jnp.float32),
    )(a2, dr, b2)


def kernel(x, edge_index, W1, b1, W2, b2):
    f32 = jnp.float32
    src = edge_index[0]
    dst = edge_index[1]

    npad_e = _EPAD - _E
    i = jnp.arange(npad_e, dtype=jnp.int32)
    src_p = jnp.concatenate([src, _N + (i % _TRASH)])
    dst_p = jnp.concatenate([dst, _N + ((i * 7 + 3) % _TRASH)])
    src_deg = src_p.reshape(_NC * _NS, _DEG_NCHUNK, _CHUNK)
    dst_deg = dst_p.reshape(_NC * _NS, _DEG_NCHUNK, _CHUNK)
    src_agg = src_p.reshape(_NS, _AGG_NCHUNK, _CHUNK)
    dst_agg = dst_p.reshape(_NS, _AGG_NCHUNK, _CHUNK)

    ones_tile = jnp.ones((_CHUNK, _W), f32)
    zeros1 = jnp.zeros((_RPT, _W), f32)
    zerosd = jnp.zeros((_RPT, _HALF), f32)

    h0, h1 = _sc_hist(dst_deg, ones_tile, zeros1)
    dr_w = _tc_rsqrt_sum(h0, h1)
    dr = dr_w[:, :1]

    c0, c1 = _sc_gather_scatter_w1(dr_w, dst_deg, src_deg, zeros1)

    xpad = jnp.concatenate([x, jnp.zeros((_TRASH, _D_IN), f32)], axis=0)
    xs, dc = _tc_colscale(c0, c1, xpad)

    a0, a1h = _sc_agg(xs[:, :_HALF], xs[:, _HALF:], src_agg, dst_agg, zerosd)
    agg1 = jnp.concatenate([a0, a1h], axis=1)

    m2 = _tc_mlp(agg1, dr, W1, b1.reshape(1, _D_HID), W2, dc)

    g0, g1 = _sc_agg(m2[:, :_HALF], m2[:, _HALF:], src_agg, dst_agg, zerosd)
    agg2 = jnp.concatenate([g0, g1], axis=1)

    out = _tc_logsoftmax(agg2, dr, b2.reshape(1, _D_OUT))
    return out[:_N]


# speedup vs baseline: 9.9883x; 1.2938x over previous
import functools

import jax
import jax.numpy as jnp
from jax import lax
from jax.experimental import pallas as pl
from jax.experimental.pallas import tpu as pltpu
from jax.experimental.pallas import tpu_sc as plsc

_N = 10000
_E = 160000
_D_IN = 256
_D_HID = 512
_D_OUT = 256
_NC = 2
_NS = 16
_CHUNK = 128
_NPAD = 10112
_RPT = _NPAD // _NS
_TRASH = _NPAD - _N
_EPAD = 163840
_DEG_NCHUNK = _EPAD // (_NC * _NS * _CHUNK)
_AGG_NCHUNK = _EPAD // (_NS * _CHUNK)
_AGG_HNCH = _AGG_NCHUNK // 2
_HALF = 128
_W = 16

_SC_PARAMS = pltpu.CompilerParams(use_tc_tiling_on_sc=False)


def _mesh():
    return plsc.VectorSubcoreMesh(
        core_axis_name="c", subcore_axis_name="s", num_cores=_NC, num_subcores=_NS
    )


def _sc_hist(sidx, ones_tile, zeros1):
    @functools.partial(
        pl.kernel,
        mesh=_mesh(),
        out_type=[jax.ShapeDtypeStruct((_NPAD, _W), jnp.float32) for _ in range(2)],
        scratch_types=[
            pltpu.VMEM((_DEG_NCHUNK, _CHUNK), jnp.int32),
            pltpu.VMEM((_CHUNK, _W), jnp.float32),
            pltpu.VMEM_SHARED((_NPAD, _W), jnp.float32),
        ],
        compiler_params=_SC_PARAMS,
    )
    def k(sidx_h, ones_h, z_h, o0_h, o1_h, sv, buf, acc):
        c = lax.axis_index("c")
        s = lax.axis_index("s")
        tid = c * _NS + s
        slc = pl.ds(s * _RPT, _RPT)
        pltpu.sync_copy(z_h, acc.at[slc])
        pltpu.sync_copy(ones_h, buf)
        pltpu.sync_copy(sidx_h.at[tid], sv)
        plsc.subcore_barrier()

        @pl.loop(0, _DEG_NCHUNK)
        def _(j):
            pltpu.sync_copy(buf, acc.at[sv.at[j]], add=True)

        plsc.subcore_barrier()

        @pl.when(c == 0)
        def _():
            pltpu.sync_copy(acc.at[slc], o0_h.at[slc])

        @pl.when(c == 1)
        def _():
            pltpu.sync_copy(acc.at[slc], o1_h.at[slc])

    return k(sidx, ones_tile, zeros1)


def _sc_gather_scatter_w1(table, gidx, sidx, zeros1):
    @functools.partial(
        pl.kernel,
        mesh=_mesh(),
        out_type=[jax.ShapeDtypeStruct((_NPAD, _W), jnp.float32) for _ in range(2)],
        scratch_types=[
            pltpu.VMEM((_DEG_NCHUNK, _CHUNK), jnp.int32),
            pltpu.VMEM((_DEG_NCHUNK, _CHUNK), jnp.int32),
            pltpu.VMEM((_CHUNK, _W), jnp.float32),
            pltpu.VMEM((_CHUNK, _W), jnp.float32),
            pltpu.VMEM_SHARED((_NPAD, _W), jnp.float32),
            pltpu.SemaphoreType.DMA,
            pltpu.SemaphoreType.DMA,
        ],
        compiler_params=_SC_PARAMS,
    )
    def k(table_h, gidx_h, sidx_h, z_h, o0_h, o1_h, gv, sv, buf0, buf1, acc, sem0, sem1):
        c = lax.axis_index("c")
        s = lax.axis_index("s")
        tid = c * _NS + s
        slc = pl.ds(s * _RPT, _RPT)
        pltpu.sync_copy(z_h, acc.at[slc])
        pltpu.sync_copy(gidx_h.at[tid], gv)
        pltpu.sync_copy(sidx_h.at[tid], sv)
        plsc.subcore_barrier()

        pltpu.async_copy(table_h.at[gv.at[0]], buf0, sem0)

        @pl.loop(0, _DEG_NCHUNK, step=2)
        def _(j):
            pltpu.async_copy(table_h.at[gv.at[j + 1]], buf1, sem1)
            pltpu.make_async_copy(table_h.at[gv.at[j]], buf0, sem0).wait()
            pltpu.sync_copy(buf0, acc.at[sv.at[j]], add=True)

            @pl.when(j + 2 < _DEG_NCHUNK)
            def _():
                pltpu.async_copy(table_h.at[gv.at[j + 2]], buf0, sem0)

            pltpu.make_async_copy(table_h.at[gv.at[j + 1]], buf1, sem1).wait()
            pltpu.sync_copy(buf1, acc.at[sv.at[j + 1]], add=True)

        plsc.subcore_barrier()

        @pl.when(c == 0)
        def _():
            pltpu.sync_copy(acc.at[slc], o0_h.at[slc])

        @pl.when(c == 1)
        def _():
            pltpu.sync_copy(acc.at[slc], o1_h.at[slc])

    return k(table, gidx, sidx, zeros1)


def _sc_agg(t0, t1, gidx, sidx, zerosd):
    @functools.partial(
        pl.kernel,
        mesh=_mesh(),
        out_type=[jax.ShapeDtypeStruct((_NPAD, _HALF), jnp.float32) for _ in range(2)],
        scratch_types=[
            pltpu.VMEM((_AGG_HNCH, _CHUNK), jnp.int32),
            pltpu.VMEM((_AGG_HNCH, _CHUNK), jnp.int32),
            pltpu.VMEM((_CHUNK, _HALF), jnp.float32),
            pltpu.VMEM((_CHUNK, _HALF), jnp.float32),
            pltpu.VMEM_SHARED((_NPAD, _HALF), jnp.float32),
            pltpu.SemaphoreType.DMA,
            pltpu.SemaphoreType.DMA,
        ],
    )
    def k(t0_h, t1_h, gidx_h, sidx_h, z_h, o0_h, o1_h, gv, sv, buf0, buf1, acc, sem0, sem1):
        c = lax.axis_index("c")
        s = lax.axis_index("s")
        slc = pl.ds(s * _RPT, _RPT)
        pltpu.sync_copy(z_h, acc.at[slc])
        plsc.subcore_barrier()

        def edge_loop(th):
            for p in range(2):
                pltpu.sync_copy(gidx_h.at[s].at[pl.ds(p * _AGG_HNCH, _AGG_HNCH)], gv)
                pltpu.sync_copy(sidx_h.at[s].at[pl.ds(p * _AGG_HNCH, _AGG_HNCH)], sv)
                pltpu.async_copy(th.at[gv.at[0]], buf0, sem0)

                @pl.loop(0, _AGG_HNCH, step=2)
                def _(j):
                    pltpu.async_copy(th.at[gv.at[j + 1]], buf1, sem1)
                    pltpu.make_async_copy(th.at[gv.at[j]], buf0, sem0).wait()
                    pltpu.sync_copy(buf0, acc.at[sv.at[j]], add=True)

                    @pl.when(j + 2 < _AGG_HNCH)
                    def _():
                        pltpu.async_copy(th.at[gv.at[j + 2]], buf0, sem0)

                    pltpu.make_async_copy(th.at[gv.at[j + 1]], buf1, sem1).wait()
                    pltpu.sync_copy(buf1, acc.at[sv.at[j + 1]], add=True)

        @pl.when(c == 0)
        def _():
            edge_loop(t0_h)

        @pl.when(c == 1)
        def _():
            edge_loop(t1_h)

        plsc.subcore_barrier()

        @pl.when(c == 0)
        def _():
            pltpu.sync_copy(acc.at[slc], o0_h.at[slc])

        @pl.when(c == 1)
        def _():
            pltpu.sync_copy(acc.at[slc], o1_h.at[slc])

    return k(t0, t1, gidx, sidx, zerosd)


def _tc_rsqrt_sum(a, b):
    def body(a_ref, b_ref, o_ref):
        sm = a_ref[...] + b_ref[...]
        o_ref[...] = jnp.where(sm > 0, lax.rsqrt(sm), 0.0)

    return pl.pallas_call(
        body, out_shape=jax.ShapeDtypeStruct(a.shape, jnp.float32)
    )(a, b)


def _tc_colscale(c0, c1, xpad):
    r = 128
    grid = _NPAD // r

    def body(c0_ref, c1_ref, x_ref, xs_ref, dc_ref):
        sm = c0_ref[...] + c1_ref[...]
        dc = jnp.where(sm > 0, lax.rsqrt(sm), 0.0)[:, :1]
        dc_ref[...] = dc
        xs_ref[...] = x_ref[...] * dc

    return pl.pallas_call(
        body,
        grid=(grid,),
        in_specs=[
            pl.BlockSpec((r, _W), lambda i: (i, 0)),
            pl.BlockSpec((r, _W), lambda i: (i, 0)),
            pl.BlockSpec((r, _D_IN), lambda i: (i, 0)),
        ],
        out_specs=[
            pl.BlockSpec((r, _D_IN), lambda i: (i, 0)),
            pl.BlockSpec((r, 1), lambda i: (i, 0)),
        ],
        out_shape=[
            jax.ShapeDtypeStruct((_NPAD, _D_IN), jnp.float32),
            jax.ShapeDtypeStruct((_NPAD, 1), jnp.float32),
        ],
    )(c0, c1, xpad)


def _tc_mlp(a1, dr, w1, b1, w2, dc):
    r = 128
    grid = _NPAD // r

    def body(a_ref, dr_ref, w1_ref, b1_ref, w2_ref, dc_ref, o_ref):
        a = a_ref[...] * dr_ref[...]
        h = jnp.dot(a, w1_ref[...], preferred_element_type=jnp.float32)
        h = jnp.maximum(h + b1_ref[...], 0.0)
        m2 = jnp.dot(h, w2_ref[...], preferred_element_type=jnp.float32)
        o_ref[...] = m2 * dc_ref[...]

    return pl.pallas_call(
        body,
        grid=(grid,),
        in_specs=[
            pl.BlockSpec((r, _D_IN), lambda i: (i, 0)),
            pl.BlockSpec((r, 1), lambda i: (i, 0)),
            pl.BlockSpec((_D_IN, _D_HID), lambda i: (0, 0)),
            pl.BlockSpec((1, _D_HID), lambda i: (0, 0)),
            pl.BlockSpec((_D_HID, _D_OUT), lambda i: (0, 0)),
            pl.BlockSpec((r, 1), lambda i: (i, 0)),
        ],
        out_specs=pl.BlockSpec((r, _D_OUT), lambda i: (i, 0)),
        out_shape=jax.ShapeDtypeStruct((_NPAD, _D_OUT), jnp.float32),
    )(a1, dr, w1, b1, w2, dc)


def _tc_logsoftmax(a2, dr, b2):
    r = 128
    grid = _NPAD // r

    def body(a_ref, dr_ref, b2_ref, o_ref):
        v = a_ref[...] * dr_ref[...] + b2_ref[...]
        m = jnp.max(v, axis=-1, keepdims=True)
        z = v - m
        o_ref[...] = z - jnp.log(jnp.sum(jnp.exp(z), axis=-1, keepdims=True))

    return pl.pallas_call(
        body,
        grid=(grid,),
        in_specs=[
            pl.BlockSpec((r, _D_OUT), lambda i: (i, 0)),
            pl.BlockSpec((r, 1), lambda i: (i, 0)),
            pl.BlockSpec((1, _D_OUT), lambda i: (0, 0)),
        ],
        out_specs=pl.BlockSpec((r, _D_OUT), lambda i: (i, 0)),
        out_shape=jax.ShapeDtypeStruct((_NPAD, _D_OUT), jnp.float32),
    )(a2, dr, b2)


def kernel(x, edge_index, W1, b1, W2, b2):
    f32 = jnp.float32
    src = edge_index[0]
    dst = edge_index[1]

    npad_e = _EPAD - _E
    i = jnp.arange(npad_e, dtype=jnp.int32)
    src_p = jnp.concatenate([src, _N + (i % _TRASH)])
    dst_p = jnp.concatenate([dst, _N + ((i * 7 + 3) % _TRASH)])
    src_deg = src_p.reshape(_NC * _NS, _DEG_NCHUNK, _CHUNK)
    dst_deg = dst_p.reshape(_NC * _NS, _DEG_NCHUNK, _CHUNK)
    src_agg = src_p.reshape(_NS, _AGG_NCHUNK, _CHUNK)
    dst_agg = dst_p.reshape(_NS, _AGG_NCHUNK, _CHUNK)

    ones_tile = jnp.ones((_CHUNK, _W), f32)
    zeros1 = jnp.zeros((_RPT, _W), f32)
    zerosd = jnp.zeros((_RPT, _HALF), f32)

    h0, h1 = _sc_hist(dst_deg, ones_tile, zeros1)
    dr_w = _tc_rsqrt_sum(h0, h1)
    dr = dr_w[:, :1]

    c0, c1 = _sc_gather_scatter_w1(dr_w, dst_deg, src_deg, zeros1)

    xpad = jnp.concatenate([x, jnp.zeros((_TRASH, _D_IN), f32)], axis=0)
    xs, dc = _tc_colscale(c0, c1, xpad)

    a0, a1h = _sc_agg(xs[:, :_HALF], xs[:, _HALF:], src_agg, dst_agg, zerosd)
    agg1 = jnp.concatenate([a0, a1h], axis=1)

    m2 = _tc_mlp(agg1, dr, W1, b1.reshape(1, _D_HID), W2, dc)

    g0, g1 = _sc_agg(m2[:, :_HALF], m2[:, _HALF:], src_agg, dst_agg, zerosd)
    agg2 = jnp.concatenate([g0, g1], axis=1)

    out = _tc_logsoftmax(agg2, dr, b2.reshape(1, _D_OUT))
    return out[:_N]
